# Initial kernel scaffold; baseline (speedup 1.0000x reference)
import functools

import jax
import jax.numpy as jnp
from jax import lax
from jax.experimental import pallas as pl
from jax.experimental.pallas import tpu as pltpu
from jax.experimental.pallas import tpu_sc as plsc

_EPS = 1e-15
_D = 128
_N_EDGES = 320000
_TOTAL = 2 * _N_EDGES
_NC = 2
_NS = 16
_NW = _NC * _NS
_PER_W = _TOTAL // _NW
_BLK = 4000
_NBLK = _PER_W // _BLK
_SB = 80
_NSTEP = _BLK // _SB
_LANES = 16
_DCH = _D // _LANES

_ROWS = _TOTAL // _D
_POS_ROWS = _N_EDGES // _D


def _dot_row(ri, rj, e):
    p = [ri[e, pl.ds(d * _LANES, _LANES)] * rj[e, pl.ds(d * _LANES, _LANES)]
         for d in range(_DCH)]
    s0 = (p[0] + p[1]) + (p[2] + p[3])
    s1 = (p[4] + p[5]) + (p[6] + p[7])
    return jnp.sum(s0 + s1)


def _make_sc_logits():
    mesh = plsc.VectorSubcoreMesh(core_axis_name="c", subcore_axis_name="s")

    @functools.partial(
        pl.kernel,
        out_type=jax.ShapeDtypeStruct((_TOTAL,), jnp.float32),
        mesh=mesh,
        scratch_types=[
            pltpu.VMEM((_BLK,), jnp.int32),
            pltpu.VMEM((_BLK,), jnp.int32),
            pltpu.VMEM((_BLK,), jnp.float32),
            pltpu.VMEM((_SB, _D), jnp.float32),
            pltpu.VMEM((_SB, _D), jnp.float32),
            pltpu.VMEM((_SB, _D), jnp.float32),
            pltpu.VMEM((_SB, _D), jnp.float32),
            pltpu.SemaphoreType.DMA,
            pltpu.SemaphoreType.DMA,
        ],
    )
    def sc_logits(z_hbm, src_hbm, dst_hbm, out_hbm,
                  idx_i, idx_j, vals, ri0, rj0, ri1, rj1, sem0, sem1):
        wid = lax.axis_index("s") * _NC + lax.axis_index("c")
        w_base = wid * _PER_W
        slots = ((ri0, rj0, sem0), (ri1, rj1, sem1))

        def fire(t, b):
            ri, rj, sem = slots[b]
            off = pl.multiple_of(t * _SB, _SB)
            pltpu.async_copy(z_hbm.at[idx_i.at[pl.ds(off, _SB)]], ri, sem)
            pltpu.async_copy(z_hbm.at[idx_j.at[pl.ds(off, _SB)]], rj, sem)

        def drain(t, b):
            ri, rj, sem = slots[b]
            off = pl.multiple_of(t * _SB, _SB)
            pltpu.make_async_copy(z_hbm.at[idx_i.at[pl.ds(off, _SB)]], ri, sem).wait()
            pltpu.make_async_copy(z_hbm.at[idx_j.at[pl.ds(off, _SB)]], rj, sem).wait()

        def compute(t, b):
            ri, rj, _ = slots[b]
            vbase = t * _SB

            def body(ii, carry):
                e = ii * 2
                v0 = _dot_row(ri, rj, e)
                v1 = _dot_row(ri, rj, e + 1)
                vals[vbase + e] = v0
                vals[vbase + e + 1] = v1
                return carry

            lax.fori_loop(0, _SB // 2, body, 0)

        def block(blk, carry):
            bbase = pl.multiple_of(w_base + blk * _BLK, _BLK)
            pltpu.sync_copy(src_hbm.at[pl.ds(bbase, _BLK)], idx_i)
            pltpu.sync_copy(dst_hbm.at[pl.ds(bbase, _BLK)], idx_j)
            fire(0, 0)
            fire(1, 1)

            def grp(g, c):
                for b in range(2):
                    t = g * 2 + b
                    drain(t, b)
                    compute(t, b)
                    fire(t + 2, b)
                return c

            lax.fori_loop(0, _NSTEP // 2 - 1, grp, 0)
            for b in range(2):
                t = _NSTEP - 2 + b
                drain(t, b)
                compute(t, b)
            pltpu.sync_copy(vals, out_hbm.at[pl.ds(bbase, _BLK)])
            return carry

        lax.fori_loop(0, _NBLK, block, 0)

    return sc_logits


def _loss_body(v_ref, out_ref):
    v = v_ref[...]
    p = 1.0 / (1.0 + jnp.exp(-v))
    row = lax.broadcasted_iota(jnp.int32, (_ROWS, _D), 0)
    term = jnp.where(row < _POS_ROWS,
                     -jnp.log(p + _EPS),
                     -jnp.log(1.0 - p + _EPS))
    out_ref[0, 0] = jnp.sum(term) / _N_EDGES


def kernel(z, pos_edge_index, neg_edge_index):
    src = jnp.concatenate([pos_edge_index[0], neg_edge_index[0]]).astype(jnp.int32)
    dst = jnp.concatenate([pos_edge_index[1], neg_edge_index[1]]).astype(jnp.int32)
    logits = _make_sc_logits()(z, src, dst)
    loss = pl.pallas_call(
        _loss_body,
        out_shape=jax.ShapeDtypeStruct((1, 1), jnp.float32),
        out_specs=pl.BlockSpec(memory_space=pltpu.SMEM),
    )(logits.reshape(_ROWS, _D))
    return loss[0, 0]


# speedup vs baseline: 8.6635x; 8.6635x over previous
import functools

import jax
import jax.numpy as jnp
from jax import lax
from jax.experimental import pallas as pl
from jax.experimental.pallas import tpu as pltpu
from jax.experimental.pallas import tpu_sc as plsc

_EPS = 1e-15
_D = 128
_N_EDGES = 320000
_TOTAL = 2 * _N_EDGES
_NC = 2
_NS = 16
_NW = _NC * _NS
_PER_W = _TOTAL // _NW
_BLK = 4000
_NBLK = _PER_W // _BLK
_SB = 80
_NSTEP = _BLK // _SB
_LANES = 16
_DCH = _D // _LANES

_ROWS = _TOTAL // _D
_POS_ROWS = _N_EDGES // _D


def _make_sc_logits():
    mesh = plsc.VectorSubcoreMesh(core_axis_name="c", subcore_axis_name="s")

    @functools.partial(
        pl.kernel,
        out_type=jax.ShapeDtypeStruct((_TOTAL,), jnp.float32),
        mesh=mesh,
        compiler_params=pltpu.CompilerParams(needs_layout_passes=False),
        scratch_types=[
            pltpu.VMEM((_BLK,), jnp.int32),
            pltpu.VMEM((_BLK,), jnp.int32),
            pltpu.VMEM((_BLK,), jnp.float32),
            pltpu.VMEM((_SB, _D), jnp.float32),
            pltpu.VMEM((_SB, _D), jnp.float32),
            pltpu.VMEM((_SB, _D), jnp.float32),
            pltpu.VMEM((_SB, _D), jnp.float32),
            pltpu.SemaphoreType.DMA,
            pltpu.SemaphoreType.DMA,
        ],
    )
    def sc_logits(z_hbm, src_hbm, dst_hbm, out_hbm,
                  idx_i, idx_j, vals, ri0, rj0, ri1, rj1, sem0, sem1):
        wid = lax.axis_index("s") * _NC + lax.axis_index("c")
        w_base = wid * _PER_W
        slots = ((ri0, rj0, sem0), (ri1, rj1, sem1))

        def fire(t, b):
            ri, rj, sem = slots[b]
            off = pl.multiple_of(t * _SB, _SB)
            pltpu.async_copy(z_hbm.at[idx_i.at[pl.ds(off, _SB)]], ri, sem)
            pltpu.async_copy(z_hbm.at[idx_j.at[pl.ds(off, _SB)]], rj, sem)

        def drain(t, b):
            ri, rj, sem = slots[b]
            off = pl.multiple_of(t * _SB, _SB)
            pltpu.make_async_copy(z_hbm.at[idx_i.at[pl.ds(off, _SB)]], ri, sem).wait()
            pltpu.make_async_copy(z_hbm.at[idx_j.at[pl.ds(off, _SB)]], rj, sem).wait()

        iota16 = lax.iota(jnp.int32, _LANES)
        zeros_f = jnp.zeros((_LANES,), jnp.float32)

        def _dot_row(ri, rj, e):
            p = [ri[e, pl.ds(d * _LANES, _LANES)] * rj[e, pl.ds(d * _LANES, _LANES)]
                 for d in range(_DCH)]
            s0 = (p[0] + p[1]) + (p[2] + p[3])
            s1 = (p[4] + p[5]) + (p[6] + p[7])
            return jnp.sum(s0 + s1)

        def compute(t, b):
            ri, rj, _ = slots[b]
            vbase = t * _SB

            def grp_body(g, carry):
                def pair(ii, v):
                    e = g * _LANES + ii * 2
                    s0 = _dot_row(ri, rj, e)
                    s1 = _dot_row(ri, rj, e + 1)
                    v = jnp.where(iota16 == ii * 2, s0, v)
                    v = jnp.where(iota16 == ii * 2 + 1, s1, v)
                    return v

                v = lax.fori_loop(0, _LANES // 2, pair, zeros_f)
                vals[pl.ds(vbase + g * _LANES, _LANES)] = v
                return carry

            lax.fori_loop(0, _SB // _LANES, grp_body, 0)

        def block(blk, carry):
            bbase = pl.multiple_of(w_base + blk * _BLK, _BLK)
            pltpu.sync_copy(src_hbm.at[pl.ds(bbase, _BLK)], idx_i)
            pltpu.sync_copy(dst_hbm.at[pl.ds(bbase, _BLK)], idx_j)
            fire(0, 0)
            fire(1, 1)

            def grp(g, c):
                for b in range(2):
                    t = g * 2 + b
                    drain(t, b)
                    compute(t, b)
                    fire(t + 2, b)
                return c

            lax.fori_loop(0, _NSTEP // 2 - 1, grp, 0)
            for b in range(2):
                t = _NSTEP - 2 + b
                drain(t, b)
                compute(t, b)
            pltpu.sync_copy(vals, out_hbm.at[pl.ds(bbase, _BLK)])
            return carry

        lax.fori_loop(0, _NBLK, block, 0)

    return sc_logits


def _loss_body(v_ref, out_ref):
    v = v_ref[...]
    p = 1.0 / (1.0 + jnp.exp(-v))
    row = lax.broadcasted_iota(jnp.int32, (_ROWS, _D), 0)
    term = jnp.where(row < _POS_ROWS,
                     -jnp.log(p + _EPS),
                     -jnp.log(1.0 - p))
    out_ref[0, 0] = jnp.sum(term) / _N_EDGES


def kernel(z, pos_edge_index, neg_edge_index):
    src = jnp.concatenate([pos_edge_index[0], neg_edge_index[0]]).astype(jnp.int32)
    dst = jnp.concatenate([pos_edge_index[1], neg_edge_index[1]]).astype(jnp.int32)
    logits = _make_sc_logits()(z, src, dst)
    loss = pl.pallas_call(
        _loss_body,
        out_shape=jax.ShapeDtypeStruct((1, 1), jnp.float32),
        out_specs=pl.BlockSpec(memory_space=pltpu.SMEM),
    )(logits.reshape(_ROWS, _D))
    return loss[0, 0]


# speedup vs baseline: 11.1134x; 1.2828x over previous
import functools

import jax
import jax.numpy as jnp
from jax import lax
from jax.experimental import pallas as pl
from jax.experimental.pallas import tpu as pltpu
from jax.experimental.pallas import tpu_sc as plsc

_EPS = 1e-15
_D = 128
_N_EDGES = 320000
_TOTAL = 2 * _N_EDGES
_NC = 2
_NS = 16
_NW = _NC * _NS
_PER_W = _TOTAL // _NW
_BLK = 4000
_NBLK = _PER_W // _BLK
_SB = 80
_NSTEP = _BLK // _SB
_NRING = 3
_LANES = 16
_DCH = _D // _LANES

_ROWS = _TOTAL // _D
_POS_ROWS = _N_EDGES // _D


def _make_sc_logits():
    mesh = plsc.VectorSubcoreMesh(core_axis_name="c", subcore_axis_name="s")

    row_bufs = []
    for _ in range(_NRING):
        row_bufs += [pltpu.VMEM((_SB, _D), jnp.float32),
                     pltpu.VMEM((_SB, _D), jnp.float32)]

    @functools.partial(
        pl.kernel,
        out_type=jax.ShapeDtypeStruct((_TOTAL,), jnp.float32),
        mesh=mesh,
        compiler_params=pltpu.CompilerParams(needs_layout_passes=False),
        scratch_types=[
            pltpu.VMEM((_BLK,), jnp.int32),
            pltpu.VMEM((_BLK,), jnp.int32),
            pltpu.VMEM((_BLK,), jnp.float32),
            *row_bufs,
            *([pltpu.SemaphoreType.DMA] * _NRING),
        ],
    )
    def sc_logits(z_hbm, pe_hbm, ne_hbm, out_hbm,
                  idx_i, idx_j, vals, *bufs_and_sems):
        rows = bufs_and_sems[:2 * _NRING]
        sems = bufs_and_sems[2 * _NRING:]
        slots = tuple((rows[2 * b], rows[2 * b + 1], sems[b])
                      for b in range(_NRING))

        wid = lax.axis_index("s") * _NC + lax.axis_index("c")
        half = wid // _NS
        w_base = (wid % _NS) * _PER_W
        iota16 = lax.iota(jnp.int32, _LANES)
        zeros_f = jnp.zeros((_LANES,), jnp.float32)

        def fire(t, b):
            ri, rj, sem = slots[b]
            off = pl.multiple_of(t * _SB, _SB)
            pltpu.async_copy(z_hbm.at[idx_i.at[pl.ds(off, _SB)]], ri, sem)
            pltpu.async_copy(z_hbm.at[idx_j.at[pl.ds(off, _SB)]], rj, sem)

        def drain(t, b):
            ri, rj, sem = slots[b]
            off = pl.multiple_of(t * _SB, _SB)
            pltpu.make_async_copy(z_hbm.at[idx_i.at[pl.ds(off, _SB)]], ri, sem).wait()
            pltpu.make_async_copy(z_hbm.at[idx_j.at[pl.ds(off, _SB)]], rj, sem).wait()

        def _dot_row(ri, rj, e):
            p = [ri[e, pl.ds(d * _LANES, _LANES)] * rj[e, pl.ds(d * _LANES, _LANES)]
                 for d in range(_DCH)]
            s0 = (p[0] + p[1]) + (p[2] + p[3])
            s1 = (p[4] + p[5]) + (p[6] + p[7])
            return jnp.sum(s0 + s1)

        def compute(t, b):
            ri, rj, _ = slots[b]
            vbase = t * _SB

            def grp_body(g, carry):
                e0 = g * _LANES

                def quad(ii, v):
                    k0 = ii * 4
                    for k in range(4):
                        s = _dot_row(ri, rj, e0 + k0 + k)
                        v = jnp.where(iota16 == k0 + k, s, v)
                    return v

                v = lax.fori_loop(0, _LANES // 4, quad, zeros_f)
                vals[pl.ds(vbase + e0, _LANES)] = v
                return carry

            lax.fori_loop(0, _SB // _LANES, grp_body, 0)

        def block(blk, carry):
            bbase = pl.multiple_of(w_base + blk * _BLK, _BLK)

            @pl.when(half == 0)
            def _():
                pltpu.sync_copy(pe_hbm.at[pl.ds(bbase, _BLK)], idx_i)
                pltpu.sync_copy(pe_hbm.at[pl.ds(_N_EDGES + bbase, _BLK)], idx_j)

            @pl.when(half == 1)
            def _():
                pltpu.sync_copy(ne_hbm.at[pl.ds(bbase, _BLK)], idx_i)
                pltpu.sync_copy(ne_hbm.at[pl.ds(_N_EDGES + bbase, _BLK)], idx_j)

            for b in range(_NRING):
                fire(b, b)

            def grp(g, c):
                for b in range(_NRING):
                    t = g * _NRING + b
                    drain(t, b)
                    compute(t, b)
                    fire(t + _NRING, b)
                return c

            n_main = (_NSTEP - _NRING - 2) // _NRING
            lax.fori_loop(0, n_main, grp, 0)
            for t in range(n_main * _NRING, _NSTEP):
                b = t % _NRING
                drain(t, b)
                compute(t, b)
                if t + _NRING < _NSTEP:
                    fire(t + _NRING, b)
            obase = pl.multiple_of(half * _N_EDGES + bbase, _BLK)
            pltpu.sync_copy(vals, out_hbm.at[pl.ds(obase, _BLK)])
            return carry

        lax.fori_loop(0, _NBLK, block, 0)

    return sc_logits


def _loss_body(v_ref, out_ref):
    v = v_ref[...]
    p = 1.0 / (1.0 + jnp.exp(-v))
    row = lax.broadcasted_iota(jnp.int32, (_ROWS, _D), 0)
    term = jnp.where(row < _POS_ROWS,
                     -jnp.log(p + _EPS),
                     -jnp.log(1.0 - p))
    out_ref[0, 0] = jnp.sum(term) / _N_EDGES


def kernel(z, pos_edge_index, neg_edge_index):
    pe = pos_edge_index.astype(jnp.int32).reshape(-1)
    ne = neg_edge_index.astype(jnp.int32).reshape(-1)
    logits = _make_sc_logits()(z, pe, ne)
    loss = pl.pallas_call(
        _loss_body,
        out_shape=jax.ShapeDtypeStruct((1, 1), jnp.float32),
        out_specs=pl.BlockSpec(memory_space=pltpu.SMEM),
    )(logits.reshape(_ROWS, _D))
    return loss[0, 0]


# speedup vs baseline: 11.1732x; 1.0054x over previous
import functools

import jax
import jax.numpy as jnp
from jax import lax
from jax.experimental import pallas as pl
from jax.experimental.pallas import tpu as pltpu
from jax.experimental.pallas import tpu_sc as plsc

_EPS = 1e-15
_D = 128
_N_EDGES = 320000
_TOTAL = 2 * _N_EDGES
_NC = 2
_NS = 16
_NW = _NC * _NS
_PER_W = _TOTAL // _NW
_BLK = 4000
_NBLK = _PER_W // _BLK
_SB = 80
_NSTEP = _BLK // _SB
_NRING = 4
_LANES = 16
_DCH = _D // _LANES

_ROWS = _TOTAL // _D
_POS_ROWS = _N_EDGES // _D


def _make_sc_logits():
    mesh = plsc.VectorSubcoreMesh(core_axis_name="c", subcore_axis_name="s")

    row_bufs = []
    for _ in range(_NRING):
        row_bufs += [pltpu.VMEM((_SB, _D), jnp.float32),
                     pltpu.VMEM((_SB, _D), jnp.float32)]

    @functools.partial(
        pl.kernel,
        out_type=jax.ShapeDtypeStruct((_TOTAL,), jnp.float32),
        mesh=mesh,
        compiler_params=pltpu.CompilerParams(needs_layout_passes=False),
        scratch_types=[
            pltpu.VMEM((_BLK,), jnp.int32),
            pltpu.VMEM((_BLK,), jnp.int32),
            pltpu.VMEM((_BLK,), jnp.float32),
            *row_bufs,
            *([pltpu.SemaphoreType.DMA] * _NRING),
        ],
    )
    def sc_logits(z_hbm, pe_hbm, ne_hbm, out_hbm,
                  idx_i, idx_j, vals, *bufs_and_sems):
        rows = bufs_and_sems[:2 * _NRING]
        sems = bufs_and_sems[2 * _NRING:]
        slots = tuple((rows[2 * b], rows[2 * b + 1], sems[b])
                      for b in range(_NRING))

        wid = lax.axis_index("s") * _NC + lax.axis_index("c")
        half = wid // _NS
        w_base = (wid % _NS) * _PER_W
        iota16 = lax.iota(jnp.int32, _LANES)
        zeros_f = jnp.zeros((_LANES,), jnp.float32)

        def fire(t, b):
            ri, rj, sem = slots[b]
            off = pl.multiple_of(t * _SB, _SB)
            pltpu.async_copy(z_hbm.at[idx_i.at[pl.ds(off, _SB)]], ri, sem)
            pltpu.async_copy(z_hbm.at[idx_j.at[pl.ds(off, _SB)]], rj, sem)

        def drain(t, b):
            ri, rj, sem = slots[b]
            off = pl.multiple_of(t * _SB, _SB)
            pltpu.make_async_copy(z_hbm.at[idx_i.at[pl.ds(off, _SB)]], ri, sem).wait()
            pltpu.make_async_copy(z_hbm.at[idx_j.at[pl.ds(off, _SB)]], rj, sem).wait()

        def _dot_row(ri, rj, e):
            p = [ri[e, pl.ds(d * _LANES, _LANES)] * rj[e, pl.ds(d * _LANES, _LANES)]
                 for d in range(_DCH)]
            s0 = (p[0] + p[1]) + (p[2] + p[3])
            s1 = (p[4] + p[5]) + (p[6] + p[7])
            return jnp.sum(s0 + s1)

        def compute(t, b):
            ri, rj, _ = slots[b]
            vbase = t * _SB

            def grp_body(g, carry):
                e0 = g * _LANES

                def quad(ii, v):
                    k0 = ii * 4
                    for k in range(4):
                        s = _dot_row(ri, rj, e0 + k0 + k)
                        v = jnp.where(iota16 == k0 + k, s, v)
                    return v

                v = lax.fori_loop(0, _LANES // 4, quad, zeros_f)
                vals[pl.ds(vbase + e0, _LANES)] = v
                return carry

            lax.fori_loop(0, _SB // _LANES, grp_body, 0)

        def block(blk, carry):
            bbase = pl.multiple_of(w_base + blk * _BLK, _BLK)

            @pl.when(half == 0)
            def _():
                pltpu.sync_copy(pe_hbm.at[pl.ds(bbase, _BLK)], idx_i)
                pltpu.sync_copy(pe_hbm.at[pl.ds(_N_EDGES + bbase, _BLK)], idx_j)

            @pl.when(half == 1)
            def _():
                pltpu.sync_copy(ne_hbm.at[pl.ds(bbase, _BLK)], idx_i)
                pltpu.sync_copy(ne_hbm.at[pl.ds(_N_EDGES + bbase, _BLK)], idx_j)

            for b in range(_NRING):
                fire(b, b)

            def grp(g, c):
                for b in range(_NRING):
                    t = g * _NRING + b
                    drain(t, b)
                    compute(t, b)
                    fire(t + _NRING, b)
                return c

            n_main = (_NSTEP - _NRING - 2) // _NRING
            lax.fori_loop(0, n_main, grp, 0)
            for t in range(n_main * _NRING, _NSTEP):
                b = t % _NRING
                drain(t, b)
                compute(t, b)
                if t + _NRING < _NSTEP:
                    fire(t + _NRING, b)
            obase = pl.multiple_of(half * _N_EDGES + bbase, _BLK)
            pltpu.sync_copy(vals, out_hbm.at[pl.ds(obase, _BLK)])
            return carry

        lax.fori_loop(0, _NBLK, block, 0)

    return sc_logits


def _loss_body(v_ref, out_ref):
    v = v_ref[...]
    p = 1.0 / (1.0 + jnp.exp(-v))
    row = lax.broadcasted_iota(jnp.int32, (_ROWS, _D), 0)
    term = jnp.where(row < _POS_ROWS,
                     -jnp.log(p + _EPS),
                     -jnp.log(1.0 - p))
    out_ref[0, 0] = jnp.sum(term) / _N_EDGES


def kernel(z, pos_edge_index, neg_edge_index):
    pe = pos_edge_index.astype(jnp.int32).reshape(-1)
    ne = neg_edge_index.astype(jnp.int32).reshape(-1)
    logits = _make_sc_logits()(z, pe, ne)
    loss = pl.pallas_call(
        _loss_body,
        out_shape=jax.ShapeDtypeStruct((1, 1), jnp.float32),
        out_specs=pl.BlockSpec(memory_space=pltpu.SMEM),
    )(logits.reshape(_ROWS, _D))
    return loss[0, 0]
